# baseline (speedup 1.0000x reference)
import jax
import jax.numpy as jnp
from jax.experimental import pallas as pl
from jax.experimental.pallas import tpu as pltpu
import numpy as np

_SHAPE = (4, 1, 96, 128, 128)
_N = int(np.prod(_SHAPE))
_COLS = 8192
_ROWS = _N // _COLS
_BLK_ROWS = 64
_GRID = _ROWS // _BLK_ROWS

_FACTOR = 1.0
_GAMA = 1.5
_EPS = 1e-08


def _body(x_ref, y_ref, out_ref, acc_ref):
    i = pl.program_id(0)

    xb = x_ref[...]
    yb = y_ref[...]
    yf = yb.astype(jnp.float32)

    pos = yb > 0
    z = jnp.where(pos, xb, -xb)
    t = jnp.exp(-jnp.abs(z))
    r = 1.0 / (1.0 + t)
    tr = t * r
    znn = z >= 0.0
    pz = jnp.where(znn, r, tr)
    q = jnp.where(znn, tr, r)
    logw = jnp.log(pz + _EPS)
    c = -(q * jnp.sqrt(q)) * logw

    sC = jnp.sum(c)
    sA = jnp.sum(jnp.where(pos, c, 0.0))
    sM = jnp.sum(yf)

    @pl.when(i == 0)
    def _init():
        acc_ref[0] = sA
        acc_ref[1] = sC
        acc_ref[2] = sM

    @pl.when(i > 0)
    def _acc():
        acc_ref[0] += sA
        acc_ref[1] += sC
        acc_ref[2] += sM

    @pl.when(i == _GRID - 1)
    def _fin():
        A = acc_ref[0]
        B = acc_ref[1] - A
        M = acc_ref[2]
        loss = (_FACTOR * ((_N - M) / M) * A + (2.0 - _FACTOR) * B) / _N
        out_ref[0] = loss


def kernel(x, y):
    x2 = x.reshape(_ROWS, _COLS)
    y2 = y.reshape(_ROWS, _COLS)
    out = pl.pallas_call(
        _body,
        grid=(_GRID,),
        in_specs=[
            pl.BlockSpec((_BLK_ROWS, _COLS), lambda i: (i, 0)),
            pl.BlockSpec((_BLK_ROWS, _COLS), lambda i: (i, 0)),
        ],
        out_specs=pl.BlockSpec(memory_space=pltpu.SMEM),
        out_shape=jax.ShapeDtypeStruct((1,), jnp.float32),
        scratch_shapes=[pltpu.SMEM((3,), jnp.float32)],
    )(x2, y2)
    return out[0]


# speedup vs baseline: 1.0983x; 1.0983x over previous
import jax
import jax.numpy as jnp
from jax.experimental import pallas as pl
from jax.experimental.pallas import tpu as pltpu
import numpy as np

_SHAPE = (4, 1, 96, 128, 128)
_N = int(np.prod(_SHAPE))
_COLS = 8192
_ROWS = _N // _COLS
_BLK_ROWS = 64
_GRID = _ROWS // _BLK_ROWS

_FACTOR = 1.0
_GAMA = 1.5
_EPS = 1e-08


def _body(x_ref, y_ref, out_ref, acc_ref):
    i = pl.program_id(0)

    xb = x_ref[...]
    yb = y_ref[...]
    yf = yb.astype(jnp.float32)

    z = xb * (2.0 * yf - 1.0)
    u = jnp.exp(-z)
    pz = 1.0 / (1.0 + u)
    q = 1.0 - pz
    logw = jnp.log(pz + _EPS)
    c = (q * jnp.sqrt(q)) * logw

    @pl.when(i == 0)
    def _init():
        acc_ref[...] = jnp.zeros_like(acc_ref)

    acc_ref[0, :] += jnp.sum(yf * c, axis=0)
    acc_ref[1, :] += jnp.sum(c, axis=0)
    acc_ref[2, :] += jnp.sum(yf, axis=0)

    @pl.when(i == _GRID - 1)
    def _fin():
        A = -jnp.sum(acc_ref[0, :])
        C = -jnp.sum(acc_ref[1, :])
        B = C - A
        M = jnp.sum(acc_ref[2, :])
        loss = (_FACTOR * ((_N - M) / M) * A + (2.0 - _FACTOR) * B) / _N
        out_ref[0] = loss


def kernel(x, y):
    x2 = x.reshape(_ROWS, _COLS)
    y2 = y.reshape(_ROWS, _COLS)
    out = pl.pallas_call(
        _body,
        grid=(_GRID,),
        in_specs=[
            pl.BlockSpec((_BLK_ROWS, _COLS), lambda i: (i, 0)),
            pl.BlockSpec((_BLK_ROWS, _COLS), lambda i: (i, 0)),
        ],
        out_specs=pl.BlockSpec(memory_space=pltpu.SMEM),
        out_shape=jax.ShapeDtypeStruct((1,), jnp.float32),
        scratch_shapes=[pltpu.VMEM((3, _COLS), jnp.float32)],
    )(x2, y2)
    return out[0]


# speedup vs baseline: 2.5565x; 2.3276x over previous
import jax
import jax.numpy as jnp
from jax.experimental import pallas as pl
from jax.experimental.pallas import tpu as pltpu
import numpy as np

_SHAPE = (4, 1, 96, 128, 128)
_N = int(np.prod(_SHAPE))
_COLS = 128
_ROWS = _N // _COLS
_BLK_ROWS = 2048
_GRID = _ROWS // _BLK_ROWS

_FACTOR = 1.0
_GAMA = 1.5
_EPS = 1e-08


def _body(x_ref, y_ref, out_ref, acc_ref):
    i = pl.program_id(0)

    xb = x_ref[...]
    yb = y_ref[...]
    yf = yb.astype(jnp.float32)

    z = xb * (2.0 * yf - 1.0)
    u = jnp.exp(-z)
    pz = 1.0 / (1.0 + u)
    q = 1.0 - pz
    logw = jnp.log(pz + _EPS)
    c = (q * jnp.sqrt(q)) * logw

    @pl.when(i == 0)
    def _init():
        acc_ref[...] = jnp.zeros_like(acc_ref)

    acc_ref[0, :] += jnp.sum(yf * c, axis=0)
    acc_ref[1, :] += jnp.sum(c, axis=0)
    acc_ref[2, :] += jnp.sum(yf, axis=0)

    @pl.when(i == _GRID - 1)
    def _fin():
        A = -jnp.sum(acc_ref[0, :])
        C = -jnp.sum(acc_ref[1, :])
        B = C - A
        M = jnp.sum(acc_ref[2, :])
        loss = (_FACTOR * ((_N - M) / M) * A + (2.0 - _FACTOR) * B) / _N
        out_ref[0] = loss


def kernel(x, y):
    x2 = x.reshape(_ROWS, _COLS)
    y2 = y.reshape(_ROWS, _COLS)
    out = pl.pallas_call(
        _body,
        grid=(_GRID,),
        in_specs=[
            pl.BlockSpec((_BLK_ROWS, _COLS), lambda i: (i, 0)),
            pl.BlockSpec((_BLK_ROWS, _COLS), lambda i: (i, 0)),
        ],
        out_specs=pl.BlockSpec(memory_space=pltpu.SMEM),
        out_shape=jax.ShapeDtypeStruct((1,), jnp.float32),
        scratch_shapes=[pltpu.VMEM((3, _COLS), jnp.float32)],
    )(x2, y2)
    return out[0]


# speedup vs baseline: 2.5985x; 1.0165x over previous
import jax
import jax.numpy as jnp
from jax.experimental import pallas as pl
from jax.experimental.pallas import tpu as pltpu
import numpy as np

_SHAPE = (4, 1, 96, 128, 128)
_N = int(np.prod(_SHAPE))
_COLS = 128
_ROWS = _N // _COLS
_BLK_ROWS = 2048
_GRID = _ROWS // _BLK_ROWS

_FACTOR = 1.0
_GAMA = 1.5
_EPS = 1e-08


_CHUNK = 64
_NCHUNK = _BLK_ROWS // _CHUNK


def _body(x_ref, y_ref, out_ref, acc_ref):
    i = pl.program_id(0)

    def _step(k, carry):
        sA, sC, sM = carry
        xb = x_ref[pl.ds(k * _CHUNK, _CHUNK), :]
        yb = y_ref[pl.ds(k * _CHUNK, _CHUNK), :]
        yf = yb.astype(jnp.float32)
        z = xb * (2.0 * yf - 1.0)
        u = jnp.exp(-z)
        pz = 1.0 / (1.0 + u)
        q = 1.0 - pz
        logw = jnp.log(pz + _EPS)
        c = (q * jnp.sqrt(q)) * logw
        return (sA + yf * c, sC + c, sM + yf)

    z0 = jnp.zeros((_CHUNK, _COLS), jnp.float32)
    sA, sC, sM = jax.lax.fori_loop(0, _NCHUNK, _step, (z0, z0, z0))

    @pl.when(i == 0)
    def _init():
        acc_ref[...] = jnp.zeros_like(acc_ref)

    acc_ref[0, :] += jnp.sum(sA, axis=0)
    acc_ref[1, :] += jnp.sum(sC, axis=0)
    acc_ref[2, :] += jnp.sum(sM, axis=0)

    @pl.when(i == _GRID - 1)
    def _fin():
        A = -jnp.sum(acc_ref[0, :])
        C = -jnp.sum(acc_ref[1, :])
        B = C - A
        M = jnp.sum(acc_ref[2, :])
        loss = (_FACTOR * ((_N - M) / M) * A + (2.0 - _FACTOR) * B) / _N
        out_ref[0] = loss


def kernel(x, y):
    x2 = x.reshape(_ROWS, _COLS)
    y2 = y.reshape(_ROWS, _COLS)
    out = pl.pallas_call(
        _body,
        grid=(_GRID,),
        in_specs=[
            pl.BlockSpec((_BLK_ROWS, _COLS), lambda i: (i, 0)),
            pl.BlockSpec((_BLK_ROWS, _COLS), lambda i: (i, 0)),
        ],
        out_specs=pl.BlockSpec(memory_space=pltpu.SMEM),
        out_shape=jax.ShapeDtypeStruct((1,), jnp.float32),
        scratch_shapes=[pltpu.VMEM((3, _COLS), jnp.float32)],
    )(x2, y2)
    return out[0]


# speedup vs baseline: 3.1557x; 1.2144x over previous
import jax
import jax.numpy as jnp
from jax.experimental import pallas as pl
from jax.experimental.pallas import tpu as pltpu
import numpy as np

_SHAPE = (4, 1, 96, 128, 128)
_N = int(np.prod(_SHAPE))
_COLS = 128
_ROWS = _N // _COLS
_BLK_ROWS = 2048
_GRID = _ROWS // _BLK_ROWS

_FACTOR = 1.0
_GAMA = 1.5
_EPS = 1e-08


_CHUNK = 64
_NCHUNK = _BLK_ROWS // _CHUNK


def _body(x_ref, y_ref, out_ref, acc_ref):
    i = pl.program_id(0)

    def _step(k, carry):
        sA, sC, sM = carry
        xb = x_ref[pl.ds(k * _CHUNK, _CHUNK), :]
        yb = y_ref[pl.ds(k * _CHUNK, _CHUNK), :]
        yf = yb.astype(jnp.float32)
        z = xb * (2.0 * yf - 1.0)
        u = jnp.exp(-z)
        pz = 1.0 / (1.0 + u)
        q = 1.0 - pz
        logw = jnp.log(pz + _EPS)
        c = (q * jnp.sqrt(q)) * logw
        return (sA + yf * c, sC + c, sM + yf)

    z0 = jnp.zeros((_CHUNK, _COLS), jnp.float32)
    carry = (z0, z0, z0)
    for k in range(_NCHUNK):
        carry = _step(k, carry)
    sA, sC, sM = carry

    @pl.when(i == 0)
    def _init():
        acc_ref[...] = jnp.zeros_like(acc_ref)

    acc_ref[0, :] += jnp.sum(sA, axis=0)
    acc_ref[1, :] += jnp.sum(sC, axis=0)
    acc_ref[2, :] += jnp.sum(sM, axis=0)

    @pl.when(i == _GRID - 1)
    def _fin():
        A = -jnp.sum(acc_ref[0, :])
        C = -jnp.sum(acc_ref[1, :])
        B = C - A
        M = jnp.sum(acc_ref[2, :])
        loss = (_FACTOR * ((_N - M) / M) * A + (2.0 - _FACTOR) * B) / _N
        out_ref[0] = loss


def kernel(x, y):
    x2 = x.reshape(_ROWS, _COLS)
    y2 = y.reshape(_ROWS, _COLS)
    out = pl.pallas_call(
        _body,
        grid=(_GRID,),
        in_specs=[
            pl.BlockSpec((_BLK_ROWS, _COLS), lambda i: (i, 0)),
            pl.BlockSpec((_BLK_ROWS, _COLS), lambda i: (i, 0)),
        ],
        out_specs=pl.BlockSpec(memory_space=pltpu.SMEM),
        out_shape=jax.ShapeDtypeStruct((1,), jnp.float32),
        scratch_shapes=[pltpu.VMEM((3, _COLS), jnp.float32)],
    )(x2, y2)
    return out[0]


# speedup vs baseline: 3.3688x; 1.0675x over previous
import jax
import jax.numpy as jnp
from jax.experimental import pallas as pl
from jax.experimental.pallas import tpu as pltpu
import numpy as np

_SHAPE = (4, 1, 96, 128, 128)
_N = int(np.prod(_SHAPE))
_COLS = 128
_ROWS = _N // _COLS
_BLK_ROWS = 2048
_GRID = _ROWS // _BLK_ROWS

_FACTOR = 1.0
_GAMA = 1.5
_EPS = 1e-08


_CHUNK = 64
_NCHUNK = _BLK_ROWS // _CHUNK


def _body(x_ref, y_ref, out_ref, acc_ref):
    i = pl.program_id(0)

    def _step(k, carry):
        sA, sC, sM = carry
        xb = x_ref[pl.ds(k * _CHUNK, _CHUNK), :]
        yb = y_ref[pl.ds(k * _CHUNK, _CHUNK), :]
        yf = yb.astype(jnp.float32)
        z = xb * (2.0 * yf - 1.0)
        a = jnp.minimum(z * (-1.4426950408889634), 126.0)
        u = jnp.exp2(a)
        d = 1.0 + u
        L = jnp.log2(d)
        c = jnp.exp2(1.5 * (a - L)) * L
        return (sA + yf * c, sC + c, sM + yf)

    z0 = jnp.zeros((_CHUNK, _COLS), jnp.float32)
    carry = (z0, z0, z0)
    for k in range(_NCHUNK):
        carry = _step(k, carry)
    sA, sC, sM = carry

    @pl.when(i == 0)
    def _init():
        acc_ref[...] = jnp.zeros_like(acc_ref)

    acc_ref[0, :] += jnp.sum(sA, axis=0)
    acc_ref[1, :] += jnp.sum(sC, axis=0)
    acc_ref[2, :] += jnp.sum(sM, axis=0)

    @pl.when(i == _GRID - 1)
    def _fin():
        ln2 = 0.6931471805599453
        A = ln2 * jnp.sum(acc_ref[0, :])
        C = ln2 * jnp.sum(acc_ref[1, :])
        B = C - A
        M = jnp.sum(acc_ref[2, :])
        loss = (_FACTOR * ((_N - M) / M) * A + (2.0 - _FACTOR) * B) / _N
        out_ref[0] = loss


def kernel(x, y):
    x2 = x.reshape(_ROWS, _COLS)
    y2 = y.reshape(_ROWS, _COLS)
    out = pl.pallas_call(
        _body,
        grid=(_GRID,),
        in_specs=[
            pl.BlockSpec((_BLK_ROWS, _COLS), lambda i: (i, 0)),
            pl.BlockSpec((_BLK_ROWS, _COLS), lambda i: (i, 0)),
        ],
        out_specs=pl.BlockSpec(memory_space=pltpu.SMEM),
        out_shape=jax.ShapeDtypeStruct((1,), jnp.float32),
        scratch_shapes=[pltpu.VMEM((3, _COLS), jnp.float32)],
    )(x2, y2)
    return out[0]


# speedup vs baseline: 4.1747x; 1.2392x over previous
import jax
import jax.numpy as jnp
from jax.experimental import pallas as pl
from jax.experimental.pallas import tpu as pltpu
import numpy as np

_SHAPE = (4, 1, 96, 128, 128)
_N = int(np.prod(_SHAPE))
_COLS = 128
_ROWS = _N // _COLS
_BLK_ROWS = 4096
_GRID = _ROWS // _BLK_ROWS

_FACTOR = 1.0
_GAMA = 1.5
_EPS = 1e-08


_CHUNK = 64
_NCHUNK = _BLK_ROWS // _CHUNK


def _body(x_ref, y_ref, out_ref, acc_ref):
    i = pl.program_id(0)

    def _step(k, carry):
        sA, sC, sM = carry
        xb = x_ref[pl.ds(k * _CHUNK, _CHUNK), :]
        yb = y_ref[pl.ds(k * _CHUNK, _CHUNK), :]
        yf = yb.astype(jnp.float32)
        z = xb * (2.0 * yf - 1.0)
        a = jnp.minimum(z * (-1.4426950408889634), 126.0)
        u = jnp.exp2(a)
        d = 1.0 + u
        L = jnp.log2(d)
        c = jnp.exp2(1.5 * (a - L)) * L
        return (sA + yf * c, sC + c, sM + yf)

    z0 = jnp.zeros((_CHUNK, _COLS), jnp.float32)
    carry = (z0, z0, z0)
    for k in range(_NCHUNK):
        carry = _step(k, carry)
    sA, sC, sM = carry

    @pl.when(i == 0)
    def _init():
        acc_ref[...] = jnp.zeros_like(acc_ref)

    acc_ref[0, :] += jnp.sum(sA, axis=0)
    acc_ref[1, :] += jnp.sum(sC, axis=0)
    acc_ref[2, :] += jnp.sum(sM, axis=0)

    @pl.when(i == _GRID - 1)
    def _fin():
        ln2 = 0.6931471805599453
        A = ln2 * jnp.sum(acc_ref[0, :])
        C = ln2 * jnp.sum(acc_ref[1, :])
        B = C - A
        M = jnp.sum(acc_ref[2, :])
        loss = (_FACTOR * ((_N - M) / M) * A + (2.0 - _FACTOR) * B) / _N
        out_ref[0] = loss


def kernel(x, y):
    x2 = x.reshape(_ROWS, _COLS)
    y2 = y.reshape(_ROWS, _COLS)
    out = pl.pallas_call(
        _body,
        grid=(_GRID,),
        in_specs=[
            pl.BlockSpec((_BLK_ROWS, _COLS), lambda i: (i, 0)),
            pl.BlockSpec((_BLK_ROWS, _COLS), lambda i: (i, 0)),
        ],
        out_specs=pl.BlockSpec(memory_space=pltpu.SMEM),
        out_shape=jax.ShapeDtypeStruct((1,), jnp.float32),
        scratch_shapes=[pltpu.VMEM((3, _COLS), jnp.float32)],
    )(x2, y2)
    return out[0]


# speedup vs baseline: 4.5360x; 1.0866x over previous
import jax
import jax.numpy as jnp
from jax.experimental import pallas as pl
from jax.experimental.pallas import tpu as pltpu
import numpy as np

_SHAPE = (4, 1, 96, 128, 128)
_N = int(np.prod(_SHAPE))
_COLS = 128
_ROWS = _N // _COLS
_BLK_ROWS = 8192
_GRID = _ROWS // _BLK_ROWS

_FACTOR = 1.0
_GAMA = 1.5
_EPS = 1e-08


_CHUNK = 64
_NCHUNK = _BLK_ROWS // _CHUNK


def _body(x_ref, y_ref, out_ref, acc_ref):
    i = pl.program_id(0)

    def _step(k, carry):
        sA, sC, sM = carry
        xb = x_ref[pl.ds(k * _CHUNK, _CHUNK), :]
        yb = y_ref[pl.ds(k * _CHUNK, _CHUNK), :]
        yf = yb.astype(jnp.float32)
        z = xb * (2.0 * yf - 1.0)
        a = jnp.minimum(z * (-1.4426950408889634), 126.0)
        u = jnp.exp2(a)
        d = 1.0 + u
        L = jnp.log2(d)
        c = jnp.exp2(1.5 * (a - L)) * L
        return (sA + yf * c, sC + c, sM + yf)

    z0 = jnp.zeros((_CHUNK, _COLS), jnp.float32)
    carry = (z0, z0, z0)
    for k in range(_NCHUNK):
        carry = _step(k, carry)
    sA, sC, sM = carry

    @pl.when(i == 0)
    def _init():
        acc_ref[...] = jnp.zeros_like(acc_ref)

    acc_ref[0, :] += jnp.sum(sA, axis=0)
    acc_ref[1, :] += jnp.sum(sC, axis=0)
    acc_ref[2, :] += jnp.sum(sM, axis=0)

    @pl.when(i == _GRID - 1)
    def _fin():
        ln2 = 0.6931471805599453
        A = ln2 * jnp.sum(acc_ref[0, :])
        C = ln2 * jnp.sum(acc_ref[1, :])
        B = C - A
        M = jnp.sum(acc_ref[2, :])
        loss = (_FACTOR * ((_N - M) / M) * A + (2.0 - _FACTOR) * B) / _N
        out_ref[0] = loss


def kernel(x, y):
    x2 = x.reshape(_ROWS, _COLS)
    y2 = y.reshape(_ROWS, _COLS)
    out = pl.pallas_call(
        _body,
        grid=(_GRID,),
        in_specs=[
            pl.BlockSpec((_BLK_ROWS, _COLS), lambda i: (i, 0)),
            pl.BlockSpec((_BLK_ROWS, _COLS), lambda i: (i, 0)),
        ],
        out_specs=pl.BlockSpec(memory_space=pltpu.SMEM),
        out_shape=jax.ShapeDtypeStruct((1,), jnp.float32),
        scratch_shapes=[pltpu.VMEM((3, _COLS), jnp.float32)],
    )(x2, y2)
    return out[0]
